# baseline (speedup 1.0000x reference)
import functools

import jax
import jax.numpy as jnp
from jax import lax
from jax.experimental import pallas as pl
from jax.experimental.pallas import tpu as pltpu
from jax.experimental.pallas import tpu_sc as plsc

_EMB = 64
_POS_DIM = 32
_HID = 128
_FC1 = 256
_ACT = 4
_CHUNK = 128
_BB = 16


def _sc_gather(table, idx):
    nw, nch, c = idx.shape
    emb = table.shape[1]
    mesh = plsc.VectorSubcoreMesh(core_axis_name="c", subcore_axis_name="s")
    info = plsc.get_sparse_core_info()
    num_cores = info.num_cores

    @functools.partial(
        pl.kernel,
        mesh=mesh,
        compiler_params=pltpu.CompilerParams(use_tc_tiling_on_sc=False),
        out_type=jax.ShapeDtypeStruct((nw, nch, c, emb), jnp.float32),
        scratch_types=[
            pltpu.VMEM((nch, c), jnp.int32),
            pltpu.VMEM((c, emb), jnp.float32),
            pltpu.SemaphoreType.DMA,
        ],
    )
    def k(table_hbm, idx_hbm, out_hbm, idx_v, buf, sem):
        wid = lax.axis_index("s") * num_cores + lax.axis_index("c")
        pltpu.sync_copy(idx_hbm.at[wid], idx_v)

        def body(j, carry):
            pltpu.async_copy(table_hbm.at[idx_v.at[j]], buf, sem).wait()
            pltpu.sync_copy(buf, out_hbm.at[wid, j])
            return carry

        lax.fori_loop(0, nch, body, 0)

    return k(table, idx)


def _tc_body(g_ref, pos_ref, post_ref, cw_ref, cb_ref, w1_ref, b1_ref,
             w2_ref, b2_ref, out_ref, *, ll):
    m = pos_ref.shape[0]
    bb = m // ll
    dot = functools.partial(
        jnp.dot, preferred_element_type=jnp.float32,
        precision=jax.lax.Precision.HIGHEST)

    g = g_ref[...]
    pos = pos_ref[...]
    n_pos = post_ref.shape[0]
    oh = (pos == lax.broadcasted_iota(jnp.int32, (1, n_pos), 1)
          ).astype(jnp.float32)
    post = post_ref[...]
    cw = cw_ref[...]

    u = []
    for w in range(3):
        cw_word = cw[w, :_EMB, :]
        cw_pos = cw[w, _EMB:, :]
        pw = dot(post, cw_pos)
        u_w = dot(g, cw_word) + dot(oh, pw)
        u.append(u_w.reshape(bb, ll, _HID))

    z = jnp.zeros((bb, 1, _HID), jnp.float32)
    s_sh = jnp.concatenate([z, u[0][:, :-1, :]], axis=1)
    e_sh = jnp.concatenate([u[2][:, 1:, :], z], axis=1)
    h = jnp.maximum(u[1] + s_sh + e_sh + cb_ref[...], 0.0)
    pooled = jnp.max(h, axis=1)
    f1 = jnp.maximum(dot(pooled, w1_ref[...]) + b1_ref[...], 0.0)
    out_ref[...] = dot(f1, w2_ref[...]) + b2_ref[...]


def _tc_classify(g2, pos_idx, pos_table, conv_w, conv_b, w1, b1, w2, b2,
                 interpret=False):
    b, ll = pos_idx.shape
    n_pos, pdim = pos_table.shape
    pos2 = pos_idx.reshape(b * ll, 1)
    grid = (b // _BB,)
    return pl.pallas_call(
        functools.partial(_tc_body, ll=ll),
        grid=grid,
        in_specs=[
            pl.BlockSpec((_BB * ll, _EMB), lambda i: (i, 0)),
            pl.BlockSpec((_BB * ll, 1), lambda i: (i, 0)),
            pl.BlockSpec((n_pos, pdim), lambda i: (0, 0)),
            pl.BlockSpec((3, _EMB + pdim, _HID), lambda i: (0, 0, 0)),
            pl.BlockSpec((_HID,), lambda i: (0,)),
            pl.BlockSpec((_HID, _FC1), lambda i: (0, 0)),
            pl.BlockSpec((_FC1,), lambda i: (0,)),
            pl.BlockSpec((_FC1, _ACT), lambda i: (0, 0)),
            pl.BlockSpec((_ACT,), lambda i: (0,)),
        ],
        out_specs=pl.BlockSpec((_BB, _ACT), lambda i: (i, 0)),
        out_shape=jax.ShapeDtypeStruct((b, _ACT), jnp.float32),
        interpret=interpret,
    )(g2, pos2, pos_table, conv_w, conv_b, w1, b1, w2, b2)


def kernel(dct_in, pos_in, word_table, pos_table, conv_w, conv_b, W1, b1,
           W2, b2):
    b, _, ll = dct_in.shape
    dct_idx = dct_in.reshape(b, ll)
    pos_idx = pos_in.reshape(b, ll)

    info = plsc.get_sparse_core_info()
    nw = info.num_cores * info.num_subcores
    total = b * ll
    nch = total // (nw * _CHUNK)
    idx = dct_idx.reshape(nw, nch, _CHUNK)

    gathered = _sc_gather(word_table, idx)
    g2 = gathered.reshape(total, _EMB)

    return _tc_classify(g2, pos_idx, pos_table, conv_w, conv_b, W1, b1,
                        W2, b2)


# speedup vs baseline: 1.1888x; 1.1888x over previous
import functools

import jax
import jax.numpy as jnp
from jax import lax
from jax.experimental import pallas as pl
from jax.experimental.pallas import tpu as pltpu
from jax.experimental.pallas import tpu_sc as plsc

_EMB = 64
_HID = 128
_FC1 = 256
_ACT = 4
_CHUNK = 128
_BB = 32


def _sc_gather(table, idx):
    nw, nch, c = idx.shape
    emb = table.shape[1]
    mesh = plsc.VectorSubcoreMesh(core_axis_name="c", subcore_axis_name="s")
    info = plsc.get_sparse_core_info()
    num_cores = info.num_cores

    @functools.partial(
        pl.kernel,
        mesh=mesh,
        compiler_params=pltpu.CompilerParams(use_tc_tiling_on_sc=False),
        out_type=jax.ShapeDtypeStruct((nw, nch, c, emb), table.dtype),
        scratch_types=[
            pltpu.VMEM((nch, c), jnp.int32),
            pltpu.VMEM((c, emb), table.dtype),
            pltpu.VMEM((c, emb), table.dtype),
            pltpu.SemaphoreType.DMA,
            pltpu.SemaphoreType.DMA,
        ],
    )
    def k(table_hbm, idx_hbm, out_hbm, idx_v, buf0, buf1, sem0, sem1):
        wid = lax.axis_index("s") * num_cores + lax.axis_index("c")
        pltpu.sync_copy(idx_hbm.at[wid], idx_v)

        pltpu.async_copy(table_hbm.at[idx_v.at[0]], buf0, sem0)
        pltpu.async_copy(table_hbm.at[idx_v.at[1]], buf1, sem1)

        def drain(buf, sem):
            pltpu.make_async_copy(table_hbm.at[pl.ds(0, c)], buf, sem).wait()

        def body(jj, carry):
            j0 = jj * 2
            j1 = j0 + 1
            drain(buf0, sem0)
            pltpu.sync_copy(buf0, out_hbm.at[wid, j0])

            @pl.when(j0 + 2 < nch)
            def _():
                pltpu.async_copy(table_hbm.at[idx_v.at[j0 + 2]], buf0, sem0)

            drain(buf1, sem1)
            pltpu.sync_copy(buf1, out_hbm.at[wid, j1])

            @pl.when(j1 + 2 < nch)
            def _():
                pltpu.async_copy(table_hbm.at[idx_v.at[j1 + 2]], buf1, sem1)

            return carry

        lax.fori_loop(0, nch // 2, body, 0)

    return k(table, idx)


def _tc_body(g_ref, pos_ref, post_ref, cw_ref, cb_ref, w1_ref, b1_ref,
             w2_ref, b2_ref, out_ref, *, ll):
    m = pos_ref.shape[0]
    bb = m // ll
    dotf = functools.partial(
        jnp.dot, preferred_element_type=jnp.float32,
        precision=jax.lax.Precision.HIGHEST)
    dotb = functools.partial(jnp.dot, preferred_element_type=jnp.float32)

    g = g_ref[...]
    pos = pos_ref[...]
    n_pos = post_ref.shape[0]
    oh = (pos == lax.broadcasted_iota(jnp.int32, (1, n_pos), 1)
          ).astype(jnp.bfloat16)
    post = post_ref[...]
    cw = cw_ref[...]

    u = []
    for w in range(3):
        cw_word = cw[w, :_EMB, :].astype(jnp.bfloat16)
        cw_pos = cw[w, _EMB:, :]
        pw = dotf(post, cw_pos).astype(jnp.bfloat16)
        u_w = dotb(g, cw_word) + dotb(oh, pw)
        u.append(u_w.reshape(bb, ll, _HID))

    z = jnp.zeros((bb, 1, _HID), jnp.float32)
    s_sh = jnp.concatenate([z, u[0][:, :-1, :]], axis=1)
    e_sh = jnp.concatenate([u[2][:, 1:, :], z], axis=1)
    h = jnp.maximum(u[1] + s_sh + e_sh + cb_ref[...], 0.0)
    pooled = jnp.max(h, axis=1)
    f1 = jnp.maximum(dotf(pooled, w1_ref[...]) + b1_ref[...], 0.0)
    out_ref[...] = dotf(f1, w2_ref[...]) + b2_ref[...]


def _tc_classify(g2, pos_idx, pos_table, conv_w, conv_b, w1, b1, w2, b2,
                 interpret=False):
    b, ll = pos_idx.shape
    n_pos, pdim = pos_table.shape
    pos2 = pos_idx.reshape(b * ll, 1)
    grid = (b // _BB,)
    return pl.pallas_call(
        functools.partial(_tc_body, ll=ll),
        grid=grid,
        in_specs=[
            pl.BlockSpec((_BB * ll, _EMB), lambda i: (i, 0)),
            pl.BlockSpec((_BB * ll, 1), lambda i: (i, 0)),
            pl.BlockSpec((n_pos, pdim), lambda i: (0, 0)),
            pl.BlockSpec((3, _EMB + pdim, _HID), lambda i: (0, 0, 0)),
            pl.BlockSpec((_HID,), lambda i: (0,)),
            pl.BlockSpec((_HID, _FC1), lambda i: (0, 0)),
            pl.BlockSpec((_FC1,), lambda i: (0,)),
            pl.BlockSpec((_FC1, _ACT), lambda i: (0, 0)),
            pl.BlockSpec((_ACT,), lambda i: (0,)),
        ],
        out_specs=pl.BlockSpec((_BB, _ACT), lambda i: (i, 0)),
        out_shape=jax.ShapeDtypeStruct((b, _ACT), jnp.float32),
        interpret=interpret,
    )(g2, pos2, pos_table, conv_w, conv_b, w1, b1, w2, b2)


def kernel(dct_in, pos_in, word_table, pos_table, conv_w, conv_b, W1, b1,
           W2, b2):
    b, _, ll = dct_in.shape
    dct_idx = dct_in.reshape(b, ll)
    pos_idx = pos_in.reshape(b, ll)

    info = plsc.get_sparse_core_info()
    nw = info.num_cores * info.num_subcores
    total = b * ll
    nch = total // (nw * _CHUNK)
    idx = dct_idx.reshape(nw, nch, _CHUNK)

    table16 = word_table.astype(jnp.bfloat16)
    gathered = _sc_gather(table16, idx)
    g2 = gathered.reshape(total, _EMB)

    return _tc_classify(g2, pos_idx, pos_table, conv_w, conv_b, W1, b1,
                        W2, b2)


# speedup vs baseline: 1.5816x; 1.3304x over previous
import functools

import jax
import jax.numpy as jnp
from jax import lax
from jax.experimental import pallas as pl
from jax.experimental.pallas import tpu as pltpu
from jax.experimental.pallas import tpu_sc as plsc

_EMB = 64
_HID = 128
_FC1 = 256
_ACT = 4
_CHUNK = 128
_BB = 32


def _sc_gather(table, idx):
    nw, nch, c = idx.shape
    width = table.shape[1]
    mesh = plsc.VectorSubcoreMesh(core_axis_name="c", subcore_axis_name="s")
    info = plsc.get_sparse_core_info()
    num_cores = info.num_cores

    @functools.partial(
        pl.kernel,
        mesh=mesh,
        compiler_params=pltpu.CompilerParams(use_tc_tiling_on_sc=True),
        out_type=jax.ShapeDtypeStruct((nw, nch, c, width), jnp.float32),
        scratch_types=[
            pltpu.VMEM((nch, c), jnp.int32),
            pltpu.VMEM((c, width), jnp.float32),
            pltpu.VMEM((c, width), jnp.float32),
            pltpu.SemaphoreType.DMA,
            pltpu.SemaphoreType.DMA,
        ],
    )
    def k(table_hbm, idx_hbm, out_hbm, idx_v, buf0, buf1, sem0, sem1):
        wid = lax.axis_index("s") * num_cores + lax.axis_index("c")
        pltpu.sync_copy(idx_hbm.at[wid], idx_v)

        pltpu.async_copy(table_hbm.at[idx_v.at[0]], buf0, sem0)
        pltpu.async_copy(table_hbm.at[idx_v.at[1]], buf1, sem1)

        def drain(buf, sem):
            pltpu.make_async_copy(table_hbm.at[pl.ds(0, c)], buf, sem).wait()

        def body(jj, carry):
            j0 = jj * 2
            j1 = j0 + 1
            drain(buf0, sem0)
            pltpu.sync_copy(buf0, out_hbm.at[wid, j0])

            @pl.when(j0 + 2 < nch)
            def _():
                pltpu.async_copy(table_hbm.at[idx_v.at[j0 + 2]], buf0, sem0)

            drain(buf1, sem1)
            pltpu.sync_copy(buf1, out_hbm.at[wid, j1])

            @pl.when(j1 + 2 < nch)
            def _():
                pltpu.async_copy(table_hbm.at[idx_v.at[j1 + 2]], buf1, sem1)

            return carry

        lax.fori_loop(0, nch // 2, body, 0)

    return k(table, idx)


def _tc_body(g_ref, pos_ref, par_ref, post_ref, cw_ref, cb_ref, w1_ref,
             b1_ref, w2_ref, b2_ref, out_ref):
    bb, ll = pos_ref.shape
    m = bb * ll
    dotf = functools.partial(
        jnp.dot, preferred_element_type=jnp.float32,
        precision=jax.lax.Precision.HIGHEST)
    dotb = functools.partial(jnp.dot, preferred_element_type=jnp.float32)

    g2 = g_ref[...]
    par3 = par_ref[...][:, :, None]
    gl = g2[:, :_EMB].reshape(bb, ll, _EMB)
    gr = g2[:, _EMB:].reshape(bb, ll, _EMB)
    g = jnp.where(par3 == 1, gr, gl).reshape(m, _EMB).astype(jnp.bfloat16)

    pos3 = pos_ref[...][:, :, None]
    n_pos = post_ref.shape[0]
    oh = (pos3 == lax.broadcasted_iota(jnp.int32, (1, 1, n_pos), 2)
          ).astype(jnp.bfloat16).reshape(m, n_pos)
    post = post_ref[...]
    cw = cw_ref[...]

    u = []
    for w in range(3):
        cw_word = cw[w, :_EMB, :].astype(jnp.bfloat16)
        cw_pos = cw[w, _EMB:, :]
        pw = dotf(post, cw_pos).astype(jnp.bfloat16)
        u_w = dotb(g, cw_word) + dotb(oh, pw)
        u.append(u_w.reshape(bb, ll, _HID))

    z = jnp.zeros((bb, 1, _HID), jnp.float32)
    s_sh = jnp.concatenate([z, u[0][:, :-1, :]], axis=1)
    e_sh = jnp.concatenate([u[2][:, 1:, :], z], axis=1)
    h = jnp.maximum(u[1] + s_sh + e_sh + cb_ref[...], 0.0)
    pooled = jnp.max(h, axis=1)
    f1 = jnp.maximum(dotf(pooled, w1_ref[...]) + b1_ref[...], 0.0)
    out_ref[...] = dotf(f1, w2_ref[...]) + b2_ref[...]


def _tc_classify(g2, pos_idx, par_idx, pos_table, conv_w, conv_b, w1, b1,
                 w2, b2, interpret=False):
    b, ll = pos_idx.shape
    n_pos, pdim = pos_table.shape
    grid = (b // _BB,)
    return pl.pallas_call(
        _tc_body,
        grid=grid,
        in_specs=[
            pl.BlockSpec((_BB * ll, 2 * _EMB), lambda i: (i, 0)),
            pl.BlockSpec((_BB, ll), lambda i: (i, 0)),
            pl.BlockSpec((_BB, ll), lambda i: (i, 0)),
            pl.BlockSpec((n_pos, pdim), lambda i: (0, 0)),
            pl.BlockSpec((3, _EMB + pdim, _HID), lambda i: (0, 0, 0)),
            pl.BlockSpec((_HID,), lambda i: (0,)),
            pl.BlockSpec((_HID, _FC1), lambda i: (0, 0)),
            pl.BlockSpec((_FC1,), lambda i: (0,)),
            pl.BlockSpec((_FC1, _ACT), lambda i: (0, 0)),
            pl.BlockSpec((_ACT,), lambda i: (0,)),
        ],
        out_specs=pl.BlockSpec((_BB, _ACT), lambda i: (i, 0)),
        out_shape=jax.ShapeDtypeStruct((b, _ACT), jnp.float32),
        interpret=interpret,
    )(g2, pos_idx, par_idx, pos_table, conv_w, conv_b, w1, b1, w2, b2)


def kernel(dct_in, pos_in, word_table, pos_table, conv_w, conv_b, W1, b1,
           W2, b2):
    b, _, ll = dct_in.shape
    dct_idx = dct_in.reshape(b, ll)
    pos_idx = pos_in.reshape(b, ll)

    info = plsc.get_sparse_core_info()
    nw = info.num_cores * info.num_subcores
    total = b * ll
    nch = total // (nw * _CHUNK)
    idx2 = (dct_idx >> 1).reshape(nw, nch, _CHUNK)
    par = dct_idx & 1

    table2 = word_table.reshape(word_table.shape[0] // 2, 2 * _EMB)
    gathered = _sc_gather(table2, idx2)
    g2 = gathered.reshape(total, 2 * _EMB)

    return _tc_classify(g2, pos_idx, par, pos_table, conv_w, conv_b, W1, b1,
                        W2, b2)
